# baseline (speedup 1.0000x reference)
import jax
import jax.numpy as jnp
from jax import lax
from jax.experimental import pallas as pl
from jax.experimental.pallas import tpu as pltpu

N = 2048
XD = 128
HD = 32
GD = 32
YREP = HD + GD + 1
PAD = 128
BLK = 256
GRID = N // BLK

_DN = (((0,), (0,)), ((), ()))
_F32 = jnp.float32


def _body(a_hbm, x_hbm, t_ref, w1_ref, b1_ref, wg_ref, bg_ref,
          w00a_ref, w00b_ref, w00c_ref, b00_ref,
          w10a_ref, w10b_ref, w10c_ref, b10_ref,
          w01_ref, b01_ref, w11_ref, b11_ref,
          rep_ref, y0_ref, y1_ref,
          a_s, x_s, sems, xsem):
    copies = [
        pltpu.make_async_copy(
            a_hbm.at[pl.ds(j * BLK, BLK), :], a_s.at[j], sems.at[j])
        for j in range(GRID)
    ]
    for c in copies:
        c.start()
    xcopy = pltpu.make_async_copy(x_hbm, x_s, xsem)
    xcopy.start()

    t_col = t_ref[...]
    to = jnp.concatenate([t_col, jnp.ones((N, 1), _F32)], axis=1)

    xcopy.wait()
    phi = jax.nn.relu(
        jnp.dot(x_s[...], w1_ref[...], preferred_element_type=_F32)
        + b1_ref[...])
    xl = jnp.dot(t_col * phi, wg_ref[...],
                 preferred_element_type=_F32)

    stats = jnp.zeros((N, 2), _F32)
    for j in range(GRID):
        copies[j].wait()
        stats = stats + lax.dot_general(
            a_s[j], to[j * BLK:(j + 1) * BLK, :], _DN,
            preferred_element_type=_F32)

    dinv = lax.rsqrt(stats[:, 1:2] + 1.0)
    z = stats[:, 0:1] / stats[:, 1:2]
    bm = dinv * xl
    cagg = jnp.zeros((N, GD), _F32)
    for j in range(GRID):
        cagg = cagg + lax.dot_general(
            a_s[j], bm[j * BLK:(j + 1) * BLK, :], _DN,
            preferred_element_type=_F32)
    agg = dinv * (cagg + dinv * xl)
    rep_gnn = jax.nn.relu(agg + bg_ref[...])

    y00 = jax.nn.relu(
        jnp.dot(phi, w00a_ref[...], preferred_element_type=_F32)
        + jnp.dot(rep_gnn, w00b_ref[...], preferred_element_type=_F32)
        + z * w00c_ref[...] + b00_ref[...])
    y10 = jax.nn.relu(
        jnp.dot(phi, w10a_ref[...], preferred_element_type=_F32)
        + jnp.dot(rep_gnn, w10b_ref[...], preferred_element_type=_F32)
        + z * w10c_ref[...] + b10_ref[...])
    y0_ref[...] = jnp.dot(y00, w01_ref[...],
                          preferred_element_type=_F32) + b01_ref[...]
    y1_ref[...] = jnp.dot(y10, w11_ref[...],
                          preferred_element_type=_F32) + b11_ref[...]
    rep_ref[:, 0:HD] = phi
    rep_ref[:, HD:HD + GD] = rep_gnn
    rep_ref[:, HD + GD:YREP] = z


def _pad_lanes(w):
    return jnp.pad(w, ((0, 0), (0, PAD - w.shape[1])))


def kernel(X, A, T, W1, b1, Wg, bg, W00, b00, W10, b10, W01, b01, W11, b11):
    t_col = T.reshape(N, 1).astype(_F32)
    full = lambda a: pl.BlockSpec(a.shape, lambda: (0,) * a.ndim)

    w00a = _pad_lanes(W00[0:HD, :])
    w00b = _pad_lanes(W00[HD:HD + GD, :])
    w00c = _pad_lanes(W00[HD + GD:YREP, :])
    b00p = _pad_lanes(b00.reshape(1, YREP))
    w10a = _pad_lanes(W10[0:HD, :])
    w10b = _pad_lanes(W10[HD:HD + GD, :])
    w10c = _pad_lanes(W10[HD + GD:YREP, :])
    b10p = _pad_lanes(b10.reshape(1, YREP))
    w01p = jnp.pad(W01, ((0, PAD - YREP), (0, 0)))
    w11p = jnp.pad(W11, ((0, PAD - YREP), (0, 0)))

    vmem_args = (t_col, W1, b1.reshape(1, HD), Wg, bg.reshape(1, GD),
                 w00a, w00b, w00c, b00p,
                 w10a, w10b, w10c, b10p,
                 w01p, b01.reshape(1, 1), w11p, b11.reshape(1, 1))

    rep_post, y0, y1 = pl.pallas_call(
        _body,
        in_specs=[pl.BlockSpec(memory_space=pl.ANY),
                  pl.BlockSpec(memory_space=pl.ANY)]
        + [full(a) for a in vmem_args],
        out_specs=[pl.BlockSpec((N, YREP), lambda: (0, 0)),
                   pl.BlockSpec((N, 1), lambda: (0, 0)),
                   pl.BlockSpec((N, 1), lambda: (0, 0))],
        out_shape=[jax.ShapeDtypeStruct((N, YREP), _F32),
                   jax.ShapeDtypeStruct((N, 1), _F32),
                   jax.ShapeDtypeStruct((N, 1), _F32)],
        scratch_shapes=[pltpu.VMEM((GRID, BLK, N), _F32),
                        pltpu.VMEM((N, XD), _F32),
                        pltpu.SemaphoreType.DMA((GRID,)),
                        pltpu.SemaphoreType.DMA],
    )(A, X, *vmem_args)

    return (y0.reshape(-1), y1.reshape(-1), rep_post)


# speedup vs baseline: 1.4664x; 1.4664x over previous
import jax
import jax.numpy as jnp
from jax import lax
from jax.experimental import pallas as pl
from jax.experimental.pallas import tpu as pltpu

N = 2048
XD = 128
HD = 32
GD = 32
YREP = HD + GD + 1
BLK = 256
GRID = N // BLK

_DN = (((0,), (0,)), ((), ()))
_F32 = jnp.float32


def _body(a_hbm, x_ref, t_ref, w1_ref, b1_ref, wg_ref, bg_ref,
          w00_ref, b00_ref, w10_ref, b10_ref, w01_ref, b01_ref,
          w11_ref, b11_ref,
          out_ref,
          a_s, sems):
    copies = [
        pltpu.make_async_copy(
            a_hbm.at[pl.ds(j * BLK, BLK), :], a_s.at[j], sems.at[j])
        for j in range(GRID)
    ]
    for c in copies:
        c.start()

    t_col = t_ref[...]
    phi = jax.nn.relu(
        jnp.dot(x_ref[...], w1_ref[...], preferred_element_type=_F32)
        + b1_ref[...])
    xl = jnp.dot(t_col * phi, wg_ref[...],
                 preferred_element_type=_F32)

    stats = jnp.zeros((N, 2), _F32)
    for j in range(GRID):
        copies[j].wait()
        to_blk = jnp.concatenate(
            [t_col[j * BLK:(j + 1) * BLK, :],
             jnp.ones((BLK, 1), _F32)], axis=1)
        stats = stats + lax.dot_general(
            a_s[j], to_blk, _DN, preferred_element_type=_F32)

    dinv = lax.rsqrt(stats[:, 1:2] + 1.0)
    z = stats[:, 0:1] / stats[:, 1:2]
    bm = dinv * xl
    cagg = jnp.zeros((N, GD), _F32)
    for j in range(GRID):
        cagg = cagg + lax.dot_general(
            a_s[j], bm[j * BLK:(j + 1) * BLK, :], _DN,
            preferred_element_type=_F32)
    agg = dinv * (cagg + dinv * xl)
    rep_gnn = jax.nn.relu(agg + bg_ref[...])
    rep = jnp.concatenate([phi, rep_gnn, z], axis=1)
    y00 = jax.nn.relu(
        jnp.dot(rep, w00_ref[...], preferred_element_type=_F32)
        + b00_ref[...])
    y10 = jax.nn.relu(
        jnp.dot(rep, w10_ref[...], preferred_element_type=_F32)
        + b10_ref[...])
    y0c = jnp.dot(y00, w01_ref[...],
                  preferred_element_type=_F32) + b01_ref[...]
    y1c = jnp.dot(y10, w11_ref[...],
                  preferred_element_type=_F32) + b11_ref[...]
    out_ref[...] = jnp.concatenate([rep, y0c, y1c], axis=1)


def kernel(X, A, T, W1, b1, Wg, bg, W00, b00, W10, b10, W01, b01, W11, b11):
    t_col = T.reshape(N, 1).astype(_F32)
    full = lambda a: pl.BlockSpec(a.shape, lambda: (0,) * a.ndim)

    vmem_args = (X, t_col, W1, b1.reshape(1, HD), Wg,
                 bg.reshape(1, GD), W00, b00.reshape(1, YREP),
                 W10, b10.reshape(1, YREP), W01, b01.reshape(1, 1),
                 W11, b11.reshape(1, 1))

    out = pl.pallas_call(
        _body,
        in_specs=[pl.BlockSpec(memory_space=pl.ANY)]
        + [full(a) for a in vmem_args],
        out_specs=pl.BlockSpec((N, YREP + 2), lambda: (0, 0)),
        out_shape=jax.ShapeDtypeStruct((N, YREP + 2), _F32),
        scratch_shapes=[pltpu.VMEM((GRID, BLK, N), _F32),
                        pltpu.SemaphoreType.DMA((GRID,))],
    )(A, *vmem_args)

    return (out[:, YREP], out[:, YREP + 1], out[:, :YREP])
